# baseline (speedup 1.0000x reference)
import functools

import jax
import jax.numpy as jnp
from jax import lax
from jax.experimental import pallas as pl
from jax.experimental.pallas import tpu as pltpu
from jax.experimental.pallas import tpu_sc as plsc

USER_NUM = 52643
ITEM_NUM = 91599
D = 64
B = 16384

NW = 32
BPW = B // NW
CHUNK = 128
NCHUNK = BPW // CHUNK
GROUPS = BPW // 16

_mesh = plsc.VectorSubcoreMesh(core_axis_name="c", subcore_axis_name="s")


@functools.partial(
    pl.kernel,
    out_type=(
        jax.ShapeDtypeStruct((B,), jnp.float32),
        jax.ShapeDtypeStruct((B,), jnp.float32),
    ),
    mesh=_mesh,
    scratch_types=dict(
        idx_u=pltpu.VMEM((NCHUNK, CHUNK), jnp.int32),
        idx_p=pltpu.VMEM((NCHUNK, CHUNK), jnp.int32),
        idx_n=pltpu.VMEM((NCHUNK, CHUNK), jnp.int32),
        u_rows=pltpu.VMEM((BPW, D), jnp.float32),
        p_rows=pltpu.VMEM((BPW, D), jnp.float32),
        n_rows=pltpu.VMEM((BPW, D), jnp.float32),
        p_loc=pltpu.VMEM((BPW,), jnp.float32),
        n_loc=pltpu.VMEM((BPW,), jnp.float32),
        sem_idx=pltpu.SemaphoreType.DMA,
        sem_rows=pltpu.SemaphoreType.DMA,
    ),
    compiler_params=pltpu.CompilerParams(needs_layout_passes=False,
                                         use_tc_tiling_on_sc=False),
)
def _mf_kernel(users, positives, negatives, user_table, item_table,
               p_out, n_out, *, idx_u, idx_p, idx_n,
               u_rows, p_rows, n_rows, p_loc, n_loc, sem_idx, sem_rows):
    wid = lax.axis_index("s") * 2 + lax.axis_index("c")
    base = wid * BPW

    idx_copies = []
    for j in range(NCHUNK):
        for src, dst in ((users, idx_u), (positives, idx_p),
                         (negatives, idx_n)):
            c = pltpu.make_async_copy(
                src.at[pl.ds(base + j * CHUNK, CHUNK)], dst.at[j], sem_idx)
            c.start()
            idx_copies.append(c)
    for c in idx_copies:
        c.wait()

    row_copies = []
    for j in range(NCHUNK):
        for tab, idx, dst in ((user_table, idx_u, u_rows),
                              (item_table, idx_p, p_rows),
                              (item_table, idx_n, n_rows)):
            c = pltpu.make_async_copy(
                tab.at[idx.at[j]], dst.at[pl.ds(j * CHUNK, CHUNK)], sem_rows)
            c.start()
            row_copies.append(c)
    for c in row_copies:
        c.wait()

    lane = lax.iota(jnp.int32, 16)

    def group_body(g, carry):
        rows = g * 16 + lane
        accp = jnp.zeros((16,), jnp.float32)
        accn = jnp.zeros((16,), jnp.float32)
        for d in range(D):
            dcol = jnp.full((16,), d, jnp.int32)
            u = plsc.load_gather(u_rows, [rows, dcol])
            pv = plsc.load_gather(p_rows, [rows, dcol])
            nv = plsc.load_gather(n_rows, [rows, dcol])
            accp = accp + u * pv
            accn = accn + u * nv
        p_loc[pl.ds(g * 16, 16)] = accp
        n_loc[pl.ds(g * 16, 16)] = accn
        return carry

    lax.fori_loop(0, GROUPS, group_body, 0)

    pltpu.sync_copy(p_loc, p_out.at[pl.ds(base, BPW)])
    pltpu.sync_copy(n_loc, n_out.at[pl.ds(base, BPW)])


def kernel(users, positives, negatives, user_table, item_table):
    return _mf_kernel(users.astype(jnp.int32), positives.astype(jnp.int32),
                      negatives.astype(jnp.int32), user_table, item_table)


# speedup vs baseline: 1.3077x; 1.3077x over previous
import functools

import jax
import jax.numpy as jnp
from jax import lax
from jax.experimental import pallas as pl
from jax.experimental.pallas import tpu as pltpu
from jax.experimental.pallas import tpu_sc as plsc

USER_NUM = 52643
ITEM_NUM = 91599
D = 64
B = 16384

NW = 32
BPW = B // NW
CHUNK = 128
NCHUNK = BPW // CHUNK
GROUPS = BPW // 16

_mesh = plsc.VectorSubcoreMesh(core_axis_name="c", subcore_axis_name="s")


@functools.partial(
    pl.kernel,
    out_type=(
        jax.ShapeDtypeStruct((B,), jnp.float32),
        jax.ShapeDtypeStruct((B,), jnp.float32),
    ),
    mesh=_mesh,
    scratch_types=dict(
        idx_u=pltpu.VMEM((NCHUNK, CHUNK), jnp.int32),
        idx_p=pltpu.VMEM((NCHUNK, CHUNK), jnp.int32),
        idx_n=pltpu.VMEM((NCHUNK, CHUNK), jnp.int32),
        u_rows=pltpu.VMEM((BPW, D), jnp.float32),
        p_rows=pltpu.VMEM((BPW, D), jnp.float32),
        n_rows=pltpu.VMEM((BPW, D), jnp.float32),
        p_loc=pltpu.VMEM((BPW,), jnp.float32),
        n_loc=pltpu.VMEM((BPW,), jnp.float32),
        sem_idx=pltpu.SemaphoreType.DMA,
        sem_rows=pltpu.SemaphoreType.DMA,
    ),
    compiler_params=pltpu.CompilerParams(needs_layout_passes=False,
                                         use_tc_tiling_on_sc=False),
)
def _mf_kernel(users, positives, negatives, user_table, item_table,
               p_out, n_out, *, idx_u, idx_p, idx_n,
               u_rows, p_rows, n_rows, p_loc, n_loc, sem_idx, sem_rows):
    wid = lax.axis_index("s") * 2 + lax.axis_index("c")
    base = wid * BPW

    idx_copies = []
    for j in range(NCHUNK):
        for src, dst in ((users, idx_u), (positives, idx_p),
                         (negatives, idx_n)):
            c = pltpu.make_async_copy(
                src.at[pl.ds(base + j * CHUNK, CHUNK)], dst.at[j], sem_idx)
            c.start()
            idx_copies.append(c)
    for c in idx_copies:
        c.wait()

    row_copies = []
    for j in range(NCHUNK):
        for tab, idx, dst in ((user_table, idx_u, u_rows),
                              (item_table, idx_p, p_rows),
                              (item_table, idx_n, n_rows)):
            c = pltpu.make_async_copy(
                tab.at[idx.at[j]], dst.at[pl.ds(j * CHUNK, CHUNK)], sem_rows)
            c.start()
            row_copies.append(c)
    for c in row_copies:
        c.wait()

    lane = lax.iota(jnp.int32, 16)

    def group_body(g, carry):
        rows = g * 16 + lane
        accp = jnp.zeros((16,), jnp.float32)
        accn = jnp.zeros((16,), jnp.float32)
        for d in range(D):
            dcol = (lane + d) & (D - 1)
            u = plsc.load_gather(u_rows, [rows, dcol])
            pv = plsc.load_gather(p_rows, [rows, dcol])
            nv = plsc.load_gather(n_rows, [rows, dcol])
            accp = accp + u * pv
            accn = accn + u * nv
        p_loc[pl.ds(g * 16, 16)] = accp
        n_loc[pl.ds(g * 16, 16)] = accn
        return carry

    lax.fori_loop(0, GROUPS, group_body, 0)

    pltpu.sync_copy(p_loc, p_out.at[pl.ds(base, BPW)])
    pltpu.sync_copy(n_loc, n_out.at[pl.ds(base, BPW)])


def kernel(users, positives, negatives, user_table, item_table):
    return _mf_kernel(users.astype(jnp.int32), positives.astype(jnp.int32),
                      negatives.astype(jnp.int32), user_table, item_table)
